# Initial kernel scaffold; baseline (speedup 1.0000x reference)
import jax
import jax.numpy as jnp
from jax.experimental import pallas as pl


def kernel(x, table, pe):
    raise NotImplementedError("write your pallas kernel here")


# speedup vs baseline: 1.4428x; 1.4428x over previous
import functools
import math

import jax
import jax.numpy as jnp
from jax import lax
from jax.experimental import pallas as pl
from jax.experimental.pallas import tpu as pltpu
from jax.experimental.pallas import tpu_sc as plsc

D_MODEL = 128
LANES = 16
NUM_CORES = 2
NUM_SUBCORES = 16
NUM_WORKERS = NUM_CORES * NUM_SUBCORES
SCALE = math.sqrt(float(D_MODEL))


@functools.partial(jax.jit, static_argnames=("seq", "batch"))
def _run(x2d, table, pe2d, *, seq, batch):
    n_rows = seq * batch
    rpw = n_rows // NUM_WORKERS
    spw = rpw // batch
    n_gathers = rpw // D_MODEL

    mesh = plsc.VectorSubcoreMesh(
        core_axis_name="c", subcore_axis_name="s",
        num_cores=NUM_CORES, num_subcores=NUM_SUBCORES)

    @functools.partial(
        pl.kernel,
        out_type=jax.ShapeDtypeStruct((n_rows, D_MODEL), jnp.float32),
        mesh=mesh,
        scratch_types=[
            pltpu.VMEM((n_gathers, D_MODEL), jnp.int32),
            pltpu.VMEM((rpw, D_MODEL), jnp.float32),
            pltpu.VMEM((spw, D_MODEL), jnp.float32),
            pltpu.SemaphoreType.DMA,
        ],
    )
    def run(x_hbm, table_hbm, pe_hbm, out_hbm, idx_v, rows_v, pe_v, sem):
        wid = lax.axis_index("s") * NUM_CORES + lax.axis_index("c")
        base = wid * rpw
        pbase = wid * spw

        pltpu.sync_copy(x_hbm.at[pl.ds(wid * n_gathers, n_gathers)], idx_v)
        gathers = [
            pltpu.async_copy(
                table_hbm.at[idx_v.at[g]],
                rows_v.at[pl.ds(g * D_MODEL, D_MODEL)], sem)
            for g in range(n_gathers)
        ]
        pltpu.sync_copy(pe_hbm.at[pl.ds(pbase, spw)], pe_v)
        for g in gathers:
            g.wait()

        @pl.loop(0, spw)
        def _(s):
            pv = [pe_v[s, pl.ds(j * LANES, LANES)]
                  for j in range(D_MODEL // LANES)]
            for b in range(batch):
                r = s * batch + b
                for j in range(D_MODEL // LANES):
                    sl = pl.ds(j * LANES, LANES)
                    rows_v[r, sl] = rows_v[r, sl] * SCALE + pv[j]

        pltpu.sync_copy(rows_v, out_hbm.at[pl.ds(base, rpw)])

    return run(x2d, table, pe2d)


def kernel(x, table, pe):
    seq, batch = x.shape
    x2d = x.reshape(-1, D_MODEL)
    pe2d = pe.reshape(pe.shape[0], D_MODEL)
    out = _run(x2d, table, pe2d, seq=seq, batch=batch)
    return out.reshape(seq, batch, D_MODEL)


# speedup vs baseline: 1.4594x; 1.0115x over previous
import functools
import math

import jax
import jax.numpy as jnp
from jax import lax
from jax.experimental import pallas as pl
from jax.experimental.pallas import tpu as pltpu
from jax.experimental.pallas import tpu_sc as plsc

D_MODEL = 128
LANES = 16
NUM_CORES = 2
NUM_SUBCORES = 16
NUM_WORKERS = NUM_CORES * NUM_SUBCORES
SCALE = math.sqrt(float(D_MODEL))


@functools.partial(jax.jit, static_argnames=("seq", "batch"))
def _run(x2d, table, pe2d, *, seq, batch):
    n_rows = seq * batch
    rpw = n_rows // NUM_WORKERS
    spw = rpw // batch
    n_gathers = rpw // D_MODEL

    mesh = plsc.VectorSubcoreMesh(
        core_axis_name="c", subcore_axis_name="s",
        num_cores=NUM_CORES, num_subcores=NUM_SUBCORES)

    @functools.partial(
        pl.kernel,
        out_type=jax.ShapeDtypeStruct((n_rows, D_MODEL), jnp.float32),
        mesh=mesh,
        scratch_types=[
            pltpu.VMEM((n_gathers, D_MODEL), jnp.int32),
            pltpu.VMEM((rpw, D_MODEL), jnp.float32),
            pltpu.VMEM((spw, D_MODEL), jnp.float32),
            [pltpu.SemaphoreType.DMA] * n_gathers,
            pltpu.SemaphoreType.DMA,
        ],
    )
    def run(x_hbm, table_hbm, pe_hbm, out_hbm, idx_v, rows_v, pe_v,
            gsems, st_sem):
        wid = lax.axis_index("s") * NUM_CORES + lax.axis_index("c")
        base = wid * rpw
        pbase = wid * spw
        crows = rpw // n_gathers
        cs = spw // n_gathers

        pltpu.sync_copy(x_hbm.at[pl.ds(wid * n_gathers, n_gathers)], idx_v)
        gathers = [
            pltpu.async_copy(
                table_hbm.at[idx_v.at[g]],
                rows_v.at[pl.ds(g * crows, crows)], gsems[g])
            for g in range(n_gathers)
        ]
        pltpu.sync_copy(pe_hbm.at[pl.ds(pbase, spw)], pe_v)

        stores = []
        for g in range(n_gathers):
            gathers[g].wait()

            @pl.loop(g * cs, (g + 1) * cs)
            def _(s):
                pv = [pe_v[s, pl.ds(j * LANES, LANES)]
                      for j in range(D_MODEL // LANES)]
                for b in range(batch):
                    r = s * batch + b
                    for j in range(D_MODEL // LANES):
                        sl = pl.ds(j * LANES, LANES)
                        rows_v[r, sl] = rows_v[r, sl] * SCALE + pv[j]

            stores.append(pltpu.async_copy(
                rows_v.at[pl.ds(g * crows, crows)],
                out_hbm.at[pl.ds(base + g * crows, crows)], st_sem))
        for st in stores:
            st.wait()

    return run(x2d, table, pe2d)


def kernel(x, table, pe):
    seq, batch = x.shape
    x2d = x.reshape(-1, D_MODEL)
    pe2d = pe.reshape(pe.shape[0], D_MODEL)
    out = _run(x2d, table, pe2d, seq=seq, batch=batch)
    return out.reshape(seq, batch, D_MODEL)
